# baseline (speedup 1.0000x reference)
import jax
import jax.numpy as jnp
from jax import lax
from jax.experimental import pallas as pl
from jax.experimental.pallas import tpu as pltpu
from jax.experimental.pallas import tpu_sc as plsc

N = 576
D = 768
NW_USED = 24
B_PER_W = N // NW_USED
LANES = 16
CHUNKS_PER_ROW = D // LANES


def _emb_body(x_hbm, tok_hbm, pos_hbm, out_hbm, idx_v, tok_v, pos_v, sem_g, sem_p):
    wid = lax.axis_index("s") * 2 + lax.axis_index("c")

    @pl.when(wid < NW_USED)
    def _():
        base = wid * B_PER_W
        pltpu.sync_copy(x_hbm.at[pl.ds(base, B_PER_W)], idx_v)
        g = pltpu.async_copy(tok_hbm.at[idx_v], tok_v, sem_g)
        p = pltpu.async_copy(pos_hbm.at[pl.ds(base, B_PER_W)], pos_v, sem_p)
        g.wait()
        p.wait()

        def row_body(r, _):
            def col_body(j, _):
                sl = pl.ds(j * LANES, LANES)
                tok_v[r, sl] += pos_v[r, sl]
                return 0

            return lax.fori_loop(0, CHUNKS_PER_ROW, col_body, 0)

        lax.fori_loop(0, B_PER_W, row_body, 0)
        pltpu.sync_copy(tok_v, out_hbm.at[pl.ds(base, B_PER_W)])


def kernel(x, token_table, pos_table):
    mesh = plsc.VectorSubcoreMesh(core_axis_name="c", subcore_axis_name="s")
    run = pl.kernel(
        _emb_body,
        out_type=jax.ShapeDtypeStruct((N, D), jnp.float32),
        mesh=mesh,
        scratch_types=[
            pltpu.VMEM((B_PER_W,), jnp.int32),
            pltpu.VMEM((B_PER_W, D), jnp.float32),
            pltpu.VMEM((B_PER_W, D), jnp.float32),
            pltpu.SemaphoreType.DMA,
            pltpu.SemaphoreType.DMA,
        ],
    )
    return run(x, token_table, pos_table)


# speedup vs baseline: 1.1567x; 1.1567x over previous
import jax
import jax.numpy as jnp
from jax import lax
from jax.experimental import pallas as pl
from jax.experimental.pallas import tpu as pltpu
from jax.experimental.pallas import tpu_sc as plsc

N = 576
D = 768
NW_USED = 24
B_PER_W = N // NW_USED
LANES = 16
CHUNKS_PER_ROW = D // LANES


def _emb_body(x_hbm, tok_hbm, pos_hbm, out_hbm, idx_v, tok_v, pos_v, sem_g, sem_p):
    wid = lax.axis_index("s") * 2 + lax.axis_index("c")

    @pl.when(wid < NW_USED)
    def _():
        base = wid * B_PER_W
        pltpu.sync_copy(x_hbm.at[pl.ds(base, B_PER_W)], idx_v)
        g = pltpu.async_copy(tok_hbm.at[idx_v], tok_v, sem_g)
        p = pltpu.async_copy(pos_hbm.at[pl.ds(base, B_PER_W)], pos_v, sem_p)
        g.wait()
        p.wait()

        def row_body(r, _):
            for j in range(CHUNKS_PER_ROW):
                sl = pl.ds(j * LANES, LANES)
                tok_v[r, sl] += pos_v[r, sl]
            return 0

        lax.fori_loop(0, B_PER_W, row_body, 0)
        pltpu.sync_copy(tok_v, out_hbm.at[pl.ds(base, B_PER_W)])


def kernel(x, token_table, pos_table):
    mesh = plsc.VectorSubcoreMesh(core_axis_name="c", subcore_axis_name="s")
    run = pl.kernel(
        _emb_body,
        out_type=jax.ShapeDtypeStruct((N, D), jnp.float32),
        mesh=mesh,
        scratch_types=[
            pltpu.VMEM((B_PER_W,), jnp.int32),
            pltpu.VMEM((B_PER_W, D), jnp.float32),
            pltpu.VMEM((B_PER_W, D), jnp.float32),
            pltpu.SemaphoreType.DMA,
            pltpu.SemaphoreType.DMA,
        ],
    )
    return run(x, token_table, pos_table)


# speedup vs baseline: 1.1745x; 1.0153x over previous
import jax
import jax.numpy as jnp
from jax import lax
from jax.experimental import pallas as pl
from jax.experimental.pallas import tpu as pltpu
from jax.experimental.pallas import tpu_sc as plsc

N = 576
D = 768
LANES = 16
CHUNKS_PER_ROW = D // LANES

S_SC = 128
SC_CORES = 1
NW = SC_CORES * 16
B_PER_W = S_SC // NW
SC_BASE = N - S_SC
N_TC = N - S_SC
assert S_SC % 8 == 0 and B_PER_W % 8 == 0 and SC_BASE % 8 == 0


def _sc_body(x_hbm, tok_hbm, pos_hbm, out_hbm, idx_v, tok_v, pos_v, sem_g, sem_p):
    wid = lax.axis_index("s") * SC_CORES + lax.axis_index("c")
    base = SC_BASE + wid * B_PER_W
    pltpu.sync_copy(x_hbm.at[pl.ds(base, B_PER_W)], idx_v)
    g = pltpu.async_copy(tok_hbm.at[idx_v], tok_v, sem_g)
    p = pltpu.async_copy(pos_hbm.at[pl.ds(base, B_PER_W)], pos_v, sem_p)
    g.wait()
    p.wait()

    def row_body(r, _):
        for j in range(CHUNKS_PER_ROW):
            sl = pl.ds(j * LANES, LANES)
            tok_v[r, sl] += pos_v[r, sl]
        return 0

    lax.fori_loop(0, B_PER_W, row_body, 0)
    pltpu.sync_copy(tok_v, out_hbm.at[pl.ds(wid * B_PER_W, B_PER_W)])


def _sc_embed(x, token_table, pos_table):
    mesh = plsc.VectorSubcoreMesh(
        core_axis_name="c", subcore_axis_name="s", num_cores=SC_CORES
    )
    run = pl.kernel(
        _sc_body,
        out_type=jax.ShapeDtypeStruct((S_SC, D), jnp.float32),
        mesh=mesh,
        scratch_types=[
            pltpu.VMEM((B_PER_W,), jnp.int32),
            pltpu.VMEM((B_PER_W, D), jnp.float32),
            pltpu.VMEM((B_PER_W, D), jnp.float32),
            pltpu.SemaphoreType.DMA,
            pltpu.SemaphoreType.DMA,
        ],
    )
    return run(x, token_table, pos_table)


def _tc_body(x_ref, tok_ref, pos_ref, out_ref):
    x2d = x_ref[...]
    iota = lax.broadcasted_iota(jnp.int32, (N_TC, N), 1)
    oh = (iota == x2d).astype(jnp.float32)
    y = lax.dot_general(
        oh, tok_ref[...], (((1,), (0,)), ((), ())),
        preferred_element_type=jnp.float32,
    )
    out_ref[...] = y + pos_ref[...]


def _tc_embed(x, token_table, pos_table):
    return pl.pallas_call(
        _tc_body,
        out_shape=jax.ShapeDtypeStruct((N_TC, D), jnp.float32),
        grid=(1,),
        in_specs=[
            pl.BlockSpec((N_TC, 1), lambda i: (0, 0)),
            pl.BlockSpec((N, D), lambda i: (0, 0)),
            pl.BlockSpec((N_TC, D), lambda i: (0, 0)),
        ],
        out_specs=pl.BlockSpec((N_TC, D), lambda i: (0, 0)),
    )(x[:N_TC].reshape(N_TC, 1), token_table, pos_table)


def kernel(x, token_table, pos_table):
    sc_out = _sc_embed(x, token_table, pos_table)
    tc_out = _tc_embed(x, token_table, pos_table)
    return jnp.concatenate([tc_out, sc_out], axis=0)


# speedup vs baseline: 4.7938x; 4.0816x over previous
import jax
import jax.numpy as jnp
from jax import lax
from jax.experimental import pallas as pl
from jax.experimental.pallas import tpu as pltpu
from jax.experimental.pallas import tpu_sc as plsc

N = 576
D = 768
LANES = 16
CHUNKS_PER_ROW = D // LANES

S_SC = 0
SC_CORES = 1
NW = SC_CORES * 16
B_PER_W = max(S_SC // NW, 8)
SC_BASE = N - S_SC
N_TC = N - S_SC
assert S_SC % 8 == 0 and B_PER_W % 8 == 0 and SC_BASE % 8 == 0


def _sc_body(x_hbm, tok_hbm, pos_hbm, out_hbm, idx_v, tok_v, pos_v, sem_g, sem_p):
    wid = lax.axis_index("s") * SC_CORES + lax.axis_index("c")
    base = SC_BASE + wid * B_PER_W
    pltpu.sync_copy(x_hbm.at[pl.ds(base, B_PER_W)], idx_v)
    g = pltpu.async_copy(tok_hbm.at[idx_v], tok_v, sem_g)
    p = pltpu.async_copy(pos_hbm.at[pl.ds(base, B_PER_W)], pos_v, sem_p)
    g.wait()
    p.wait()

    def row_body(r, _):
        for j in range(CHUNKS_PER_ROW):
            sl = pl.ds(j * LANES, LANES)
            tok_v[r, sl] += pos_v[r, sl]
        return 0

    lax.fori_loop(0, B_PER_W, row_body, 0)
    pltpu.sync_copy(tok_v, out_hbm.at[pl.ds(wid * B_PER_W, B_PER_W)])


def _sc_embed(x, token_table, pos_table):
    mesh = plsc.VectorSubcoreMesh(
        core_axis_name="c", subcore_axis_name="s", num_cores=SC_CORES
    )
    run = pl.kernel(
        _sc_body,
        out_type=jax.ShapeDtypeStruct((S_SC, D), jnp.float32),
        mesh=mesh,
        scratch_types=[
            pltpu.VMEM((B_PER_W,), jnp.int32),
            pltpu.VMEM((B_PER_W, D), jnp.float32),
            pltpu.VMEM((B_PER_W, D), jnp.float32),
            pltpu.SemaphoreType.DMA,
            pltpu.SemaphoreType.DMA,
        ],
    )
    return run(x, token_table, pos_table)


def _tc_body(x_ref, tok_ref, pos_ref, out_ref):
    x2d = x_ref[...]
    iota = lax.broadcasted_iota(jnp.int32, (N_TC, N), 1)
    oh = (iota == x2d).astype(jnp.float32)
    y = lax.dot_general(
        oh, tok_ref[...], (((1,), (0,)), ((), ())),
        preferred_element_type=jnp.float32,
    )
    out_ref[...] = y + pos_ref[...]


def _tc_embed(x, token_table, pos_table):
    return pl.pallas_call(
        _tc_body,
        out_shape=jax.ShapeDtypeStruct((N_TC, D), jnp.float32),
        grid=(1,),
        in_specs=[
            pl.BlockSpec((N_TC, 1), lambda i: (0, 0)),
            pl.BlockSpec((N, D), lambda i: (0, 0)),
            pl.BlockSpec((N_TC, D), lambda i: (0, 0)),
        ],
        out_specs=pl.BlockSpec((N_TC, D), lambda i: (0, 0)),
    )(x[:N_TC].reshape(N_TC, 1), token_table, pos_table)


def kernel(x, token_table, pos_table):
    return _tc_embed(x, token_table, pos_table)


# speedup vs baseline: 6.5325x; 1.3627x over previous
import jax
import jax.numpy as jnp
from jax import lax
from jax.experimental import pallas as pl
from jax.experimental.pallas import tpu as pltpu
from jax.experimental.pallas import tpu_sc as plsc

N = 576
D = 768
LANES = 16
CHUNKS_PER_ROW = D // LANES

S_SC = 0
SC_CORES = 1
NW = SC_CORES * 16
B_PER_W = max(S_SC // NW, 8)
SC_BASE = N - S_SC
N_TC = N - S_SC
assert S_SC % 8 == 0 and B_PER_W % 8 == 0 and SC_BASE % 8 == 0


def _sc_body(x_hbm, tok_hbm, pos_hbm, out_hbm, idx_v, tok_v, pos_v, sem_g, sem_p):
    wid = lax.axis_index("s") * SC_CORES + lax.axis_index("c")
    base = SC_BASE + wid * B_PER_W
    pltpu.sync_copy(x_hbm.at[pl.ds(base, B_PER_W)], idx_v)
    g = pltpu.async_copy(tok_hbm.at[idx_v], tok_v, sem_g)
    p = pltpu.async_copy(pos_hbm.at[pl.ds(base, B_PER_W)], pos_v, sem_p)
    g.wait()
    p.wait()

    def row_body(r, _):
        for j in range(CHUNKS_PER_ROW):
            sl = pl.ds(j * LANES, LANES)
            tok_v[r, sl] += pos_v[r, sl]
        return 0

    lax.fori_loop(0, B_PER_W, row_body, 0)
    pltpu.sync_copy(tok_v, out_hbm.at[pl.ds(wid * B_PER_W, B_PER_W)])


def _sc_embed(x, token_table, pos_table):
    mesh = plsc.VectorSubcoreMesh(
        core_axis_name="c", subcore_axis_name="s", num_cores=SC_CORES
    )
    run = pl.kernel(
        _sc_body,
        out_type=jax.ShapeDtypeStruct((S_SC, D), jnp.float32),
        mesh=mesh,
        scratch_types=[
            pltpu.VMEM((B_PER_W,), jnp.int32),
            pltpu.VMEM((B_PER_W, D), jnp.float32),
            pltpu.VMEM((B_PER_W, D), jnp.float32),
            pltpu.SemaphoreType.DMA,
            pltpu.SemaphoreType.DMA,
        ],
    )
    return run(x, token_table, pos_table)


def _tc_body(x_ref, tok_ref, pos_ref, out_ref):
    xv = x_ref[...]
    iota = lax.broadcasted_iota(jnp.int32, (N, N_TC), 0)
    oh_t = (iota == xv[None, :]).astype(jnp.float32)
    y = lax.dot_general(
        oh_t, tok_ref[...], (((0,), (0,)), ((), ())),
        preferred_element_type=jnp.float32,
    )
    out_ref[...] = y + pos_ref[...]


def _tc_embed(x, token_table, pos_table):
    return pl.pallas_call(
        _tc_body,
        out_shape=jax.ShapeDtypeStruct((N_TC, D), jnp.float32),
        grid=(1,),
        in_specs=[
            pl.BlockSpec((N_TC,), lambda i: (0,)),
            pl.BlockSpec((N, D), lambda i: (0, 0)),
            pl.BlockSpec((N_TC, D), lambda i: (0, 0)),
        ],
        out_specs=pl.BlockSpec((N_TC, D), lambda i: (0, 0)),
    )(x if N_TC == N else x[:N_TC], token_table, pos_table)


def kernel(x, token_table, pos_table):
    return _tc_embed(x, token_table, pos_table)
